# baseline (speedup 1.0000x reference)
import math

import jax
import jax.numpy as jnp
from jax.experimental import pallas as pl
from jax.experimental.pallas import tpu as pltpu

B, Q, S, T = 8, 16, 128, 64
D = 256
H = 8
HD = D // H
STAT_K, TOKEN_K = 8, 16
NEG = -1000000.0
INV_SQRT = 1.0 / math.sqrt(HD)


def _fused_kernel(vl_ref, q_ref, sk_ref, wqs_ref, wqt_ref, wks_ref, wkt_ref,
                  tk_ref, v_ref, wv_ref, wo_ref, out_ref):
    b = pl.program_id(0)
    nt = (((1,), (1,)), ((), ()))

    q_b = q_ref[0]
    qs = jnp.dot(q_b, wqs_ref[...], preferred_element_type=jnp.float32)
    qt = jnp.dot(q_b, wqt_ref[...], preferred_element_type=jnp.float32)
    ks = jnp.dot(sk_ref[0], wks_ref[...], preferred_element_type=jnp.float32)

    wkt = wkt_ref[...]
    qk_parts = []
    sc_parts = []
    for h in range(H):
        hsl = slice(h * HD, (h + 1) * HD)
        qk_parts.append(jax.lax.dot_general(
            qt[:, hsl], wkt[:, hsl], nt, preferred_element_type=jnp.float32))
        sc_parts.append(jax.lax.dot_general(
            qs[:, hsl], ks[:, hsl], nt, preferred_element_type=jnp.float32))
    qk = jnp.concatenate(qk_parts, axis=0)
    sc = jnp.concatenate(sc_parts, axis=0) * INV_SQRT

    iota_s = jax.lax.broadcasted_iota(jnp.int32, (H * Q, S), 1)
    sc = jnp.where(iota_s < vl_ref[0, b], sc, NEG)

    selm = jnp.zeros((H * Q, S), dtype=jnp.bool_)
    cur = sc
    big = jnp.int32(2 ** 30)
    row_max = None
    for k in range(STAT_K):
        m = jnp.max(cur, axis=-1, keepdims=True)
        if k == 0:
            row_max = m
        eq = cur == m
        fidx = jnp.min(jnp.where(eq, iota_s, big), axis=-1, keepdims=True)
        oh = iota_s == fidx
        selm = jnp.logical_or(selm, oh)
        cur = jnp.where(oh, jnp.float32(-3.0e38), cur)
    e = jnp.where(selm, jnp.exp(sc - row_max), 0.0)
    w = e / jnp.sum(e, axis=-1, keepdims=True)

    tk = tk_ref[...].reshape(S * TOKEN_K, D)
    tsc = jax.lax.dot_general(tk, qk, nt,
                              preferred_element_type=jnp.float32) * INV_SQRT
    sc3 = tsc.reshape(S, TOKEN_K, H * Q)
    mx = jnp.max(sc3, axis=1, keepdims=True)
    te = jnp.exp(sc3 - mx)
    a = te / jnp.sum(te, axis=1, keepdims=True)

    cwf = (a * w.T[:, None, :]).reshape(S * TOKEN_K, H * Q)
    m_acc = jax.lax.dot_general(
        cwf, v_ref[...].reshape(S * TOKEN_K, D), (((0,), (0,)), ((), ())),
        preferred_element_type=jnp.float32)

    wv = wv_ref[...]
    parts = []
    for h in range(H):
        hsl = slice(h * HD, (h + 1) * HD)
        parts.append(jnp.dot(m_acc[h * Q:(h + 1) * Q, :], wv[:, hsl],
                             preferred_element_type=jnp.float32))
    out_pre = jnp.concatenate(parts, axis=1)
    out_ref[...] = jnp.dot(out_pre, wo_ref[...],
                           preferred_element_type=jnp.float32)[None]


def kernel(queries, stat_keys, token_keys, values, stat_valid_lens,
           W_q_stat, W_q_token, W_k_stat, W_k_token, W_v, W_o):
    vl = stat_valid_lens.reshape(1, B).astype(jnp.int32)
    t0 = (T - TOKEN_K) // TOKEN_K

    full = pl.BlockSpec((D, D), lambda b: (0, 0))
    out = pl.pallas_call(
        _fused_kernel,
        grid=(B,),
        in_specs=[
            pl.BlockSpec(memory_space=pltpu.SMEM),
            pl.BlockSpec((1, Q, D), lambda b: (b, 0, 0)),
            pl.BlockSpec((1, S, D), lambda b: (b, 0, 0)),
            full, full, full, full,
            pl.BlockSpec((S, TOKEN_K, D), lambda b: (b, t0, 0)),
            pl.BlockSpec((S, TOKEN_K, D), lambda b: (b, t0, 0)),
            full, full,
        ],
        out_specs=pl.BlockSpec((1, Q, D), lambda b: (b, 0, 0)),
        out_shape=jax.ShapeDtypeStruct((B, Q, D), jnp.float32),
    )(vl, queries, stat_keys, W_q_stat, W_q_token, W_k_stat, W_k_token,
      token_keys, values, W_v, W_o)
    return out


# speedup vs baseline: 1.3306x; 1.3306x over previous
import math

import jax
import jax.numpy as jnp
from jax.experimental import pallas as pl
from jax.experimental.pallas import tpu as pltpu

B, Q, S, T = 8, 16, 128, 64
D = 256
H = 8
HD = D // H
STAT_K, TOKEN_K = 8, 16
NEG = -1000000.0
INV_SQRT = 1.0 / math.sqrt(HD)


def _kernel(vl_ref, qf_ref, skf_ref, wqs_ref, wqt_ref, wks_ref, wkt_ref,
            tk_ref, v_ref, wv_ref, wo_ref, out_ref, qk_scr, w_scr):
    i = pl.program_id(0)
    nt = (((1,), (1,)), ((), ()))

    @pl.when(i == 0)
    def _stat():
        qf = qf_ref[...]
        qs = jnp.dot(qf, wqs_ref[...], preferred_element_type=jnp.float32)
        qt = jnp.dot(qf, wqt_ref[...], preferred_element_type=jnp.float32)
        ks = jnp.dot(skf_ref[...], wks_ref[...],
                     preferred_element_type=jnp.float32)

        wkt = wkt_ref[...]
        for h in range(H):
            hsl = slice(h * HD, (h + 1) * HD)
            qk_scr[h] = jax.lax.dot_general(
                qt[:, hsl], wkt[:, hsl], nt,
                preferred_element_type=jnp.float32)

        iota_s = jax.lax.broadcasted_iota(jnp.int32, (H * Q, S), 1)
        blocks = []
        for b in range(B):
            qs_b = qs[b * Q:(b + 1) * Q, :]
            ks_b = ks[b * S:(b + 1) * S, :]
            rows_h = [jax.lax.dot_general(
                qs_b[:, h * HD:(h + 1) * HD], ks_b[:, h * HD:(h + 1) * HD],
                nt, preferred_element_type=jnp.float32) for h in range(H)]
            sc_b = jnp.concatenate(rows_h, axis=0) * INV_SQRT
            blocks.append(jnp.where(iota_s < vl_ref[0, b], sc_b, NEG))
        sc = jnp.concatenate(blocks, axis=0)

        iota_f = jax.lax.broadcasted_iota(jnp.int32, (B * H * Q, S), 1)
        cur = sc
        big = jnp.int32(2 ** 30)
        row_max = None
        for k in range(STAT_K):
            m = jnp.max(cur, axis=-1, keepdims=True)
            if k == 0:
                row_max = m
            eq = cur == m
            fidx = jnp.min(jnp.where(eq, iota_f, big), axis=-1, keepdims=True)
            oh = iota_f == fidx
            cur = jnp.where(oh, jnp.float32(-3.0e38), cur)
        e = jnp.where(cur != sc, jnp.exp(sc - row_max), 0.0)
        w_scr[...] = e / jnp.sum(e, axis=-1, keepdims=True)

    @pl.when(i > 0)
    def _token():
        r0 = (i - 1) * H * Q
        qk = qk_scr[:, pl.ds((i - 1) * Q, Q), :].reshape(H * Q, D)
        w = w_scr[pl.ds(r0, H * Q), :]

        tk = tk_ref[...].reshape(S * TOKEN_K, D)
        tsc = jax.lax.dot_general(tk, qk, nt,
                                  preferred_element_type=jnp.float32)
        sc3 = (tsc * INV_SQRT).reshape(S, TOKEN_K, H * Q)
        mx = jnp.max(sc3, axis=1, keepdims=True)
        te = jnp.exp(sc3 - mx)
        a = te / jnp.sum(te, axis=1, keepdims=True)

        cwf = (a * w.T[:, None, :]).reshape(S * TOKEN_K, H * Q)
        m_acc = jax.lax.dot_general(
            cwf, v_ref[...].reshape(S * TOKEN_K, D), (((0,), (0,)), ((), ())),
            preferred_element_type=jnp.float32)

        wv = wv_ref[...]
        parts = []
        for h in range(H):
            hsl = slice(h * HD, (h + 1) * HD)
            parts.append(jnp.dot(m_acc[h * Q:(h + 1) * Q, :], wv[:, hsl],
                                 preferred_element_type=jnp.float32))
        out_pre = jnp.concatenate(parts, axis=1)
        out_ref[...] = jnp.dot(out_pre, wo_ref[...],
                               preferred_element_type=jnp.float32)[None]


def kernel(queries, stat_keys, token_keys, values, stat_valid_lens,
           W_q_stat, W_q_token, W_k_stat, W_k_token, W_v, W_o):
    qf = queries.reshape(B * Q, D)
    skf = stat_keys.reshape(B * S, D)
    vl = stat_valid_lens.reshape(1, B).astype(jnp.int32)
    t0 = (T - TOKEN_K) // TOKEN_K

    full = pl.BlockSpec((D, D), lambda i: (0, 0))
    prev = lambda i: jnp.maximum(i - 1, 0)
    out = pl.pallas_call(
        _kernel,
        grid=(B + 1,),
        in_specs=[
            pl.BlockSpec(memory_space=pltpu.SMEM),
            pl.BlockSpec((B * Q, D), lambda i: (0, 0)),
            pl.BlockSpec((B * S, D), lambda i: (0, 0)),
            full, full, full, full,
            pl.BlockSpec((S, TOKEN_K, D), lambda i: (prev(i), t0, 0)),
            pl.BlockSpec((S, TOKEN_K, D), lambda i: (prev(i), t0, 0)),
            full, full,
        ],
        out_specs=pl.BlockSpec((1, Q, D), lambda i: (prev(i), 0, 0)),
        out_shape=jax.ShapeDtypeStruct((B, Q, D), jnp.float32),
        scratch_shapes=[
            pltpu.VMEM((H, B * Q, D), jnp.float32),
            pltpu.VMEM((B * H * Q, S), jnp.float32),
        ],
    )(vl, qf, skf, W_q_stat, W_q_token, W_k_stat, W_k_token,
      token_keys, values, W_v, W_o)
    return out


# speedup vs baseline: 1.6485x; 1.2390x over previous
import math

import jax
import jax.numpy as jnp
from jax import lax
from jax.experimental import pallas as pl
from jax.experimental.pallas import tpu as pltpu

B, Q, S, T = 8, 16, 128, 64
D = 256
H = 8
HD = D // H
STAT_K, TOKEN_K = 8, 16
NEG = -1000000.0
INV_SQRT = 1.0 / math.sqrt(HD)
NSLOT = 3


def _kernel(vl_ref, qf_ref, skf_ref, wqs_ref, wqt_ref, wks_ref, wkt_ref,
            tk_hbm, v_hbm, wv_ref, wo_ref, out_ref,
            qk_scr, w_scr, tkbuf, vbuf, sems):
    i = pl.program_id(0)
    nt = (((1,), (1,)), ((), ()))

    def tk_copy(blk, slot):
        return pltpu.make_async_copy(
            tk_hbm.at[pl.ds(blk * S, S), pl.ds(T - TOKEN_K, TOKEN_K), :],
            tkbuf.at[slot], sems.at[0, slot])

    def v_copy(blk, slot):
        return pltpu.make_async_copy(
            v_hbm.at[pl.ds(blk * S, S), pl.ds(T - TOKEN_K, TOKEN_K), :],
            vbuf.at[slot], sems.at[1, slot])

    @pl.when(i == 0)
    def _stat():
        tk_copy(0, 0).start()
        v_copy(0, 0).start()
        tk_copy(1, 1).start()
        v_copy(1, 1).start()

        qf = qf_ref[...]
        qs = jnp.dot(qf, wqs_ref[...], preferred_element_type=jnp.float32)
        qt = jnp.dot(qf, wqt_ref[...], preferred_element_type=jnp.float32)
        ks = jnp.dot(skf_ref[...], wks_ref[...],
                     preferred_element_type=jnp.float32)

        wkt = wkt_ref[...]
        for h in range(H):
            hsl = slice(h * HD, (h + 1) * HD)
            qk_scr[h] = jax.lax.dot_general(
                qt[:, hsl], wkt[:, hsl], nt,
                preferred_element_type=jnp.float32)

        iota_s = jax.lax.broadcasted_iota(jnp.int32, (H * Q, S), 1)
        blocks = []
        for b in range(B):
            qs_b = qs[b * Q:(b + 1) * Q, :]
            ks_b = ks[b * S:(b + 1) * S, :]
            rows_h = [jax.lax.dot_general(
                qs_b[:, h * HD:(h + 1) * HD], ks_b[:, h * HD:(h + 1) * HD],
                nt, preferred_element_type=jnp.float32) for h in range(H)]
            sc_b = jnp.concatenate(rows_h, axis=0) * INV_SQRT
            blocks.append(jnp.where(iota_s < vl_ref[0, b], sc_b, NEG))
        sc = jnp.concatenate(blocks, axis=0)

        iota_f = jax.lax.broadcasted_iota(jnp.int32, (B * H * Q, S), 1)
        cur = sc
        big = jnp.int32(2 ** 30)
        row_max = None
        for k in range(STAT_K):
            m = jnp.max(cur, axis=-1, keepdims=True)
            if k == 0:
                row_max = m
            eq = cur == m
            fidx = jnp.min(jnp.where(eq, iota_f, big), axis=-1, keepdims=True)
            oh = iota_f == fidx
            cur = jnp.where(oh, jnp.float32(-3.0e38), cur)
        e = jnp.where(cur != sc, jnp.exp(sc - row_max), 0.0)
        w_scr[...] = e / jnp.sum(e, axis=-1, keepdims=True)

    @pl.when(i > 0)
    def _token():
        b = i - 1
        slot = lax.rem(b, NSLOT)

        @pl.when(b + 2 < B)
        def _prefetch():
            nslot = lax.rem(b + 2, NSLOT)
            tk_copy(b + 2, nslot).start()
            v_copy(b + 2, nslot).start()

        tk_copy(b, slot).wait()
        v_copy(b, slot).wait()

        qk = qk_scr[:, pl.ds(b * Q, Q), :].reshape(H * Q, D)
        w = w_scr[pl.ds(b * H * Q, H * Q), :]

        tk = tkbuf[slot].reshape(S * TOKEN_K, D)
        tsc = jax.lax.dot_general(tk, qk, nt,
                                  preferred_element_type=jnp.float32)
        sc3 = (tsc * INV_SQRT).reshape(S, TOKEN_K, H * Q)
        mx = jnp.max(sc3, axis=1, keepdims=True)
        te = jnp.exp(sc3 - mx)
        a = te / jnp.sum(te, axis=1, keepdims=True)

        cwf = (a * w.T[:, None, :]).reshape(S * TOKEN_K, H * Q)
        m_acc = jax.lax.dot_general(
            cwf, vbuf[slot].reshape(S * TOKEN_K, D), (((0,), (0,)), ((), ())),
            preferred_element_type=jnp.float32)

        wv = wv_ref[...]
        parts = []
        for h in range(H):
            hsl = slice(h * HD, (h + 1) * HD)
            parts.append(jnp.dot(m_acc[h * Q:(h + 1) * Q, :], wv[:, hsl],
                                 preferred_element_type=jnp.float32))
        out_pre = jnp.concatenate(parts, axis=1)
        out_ref[...] = jnp.dot(out_pre, wo_ref[...],
                               preferred_element_type=jnp.float32)[None]


def kernel(queries, stat_keys, token_keys, values, stat_valid_lens,
           W_q_stat, W_q_token, W_k_stat, W_k_token, W_v, W_o):
    qf = queries.reshape(B * Q, D)
    skf = stat_keys.reshape(B * S, D)
    vl = stat_valid_lens.reshape(1, B).astype(jnp.int32)

    full = pl.BlockSpec((D, D), lambda i: (0, 0))
    out = pl.pallas_call(
        _kernel,
        grid=(B + 1,),
        in_specs=[
            pl.BlockSpec(memory_space=pltpu.SMEM),
            pl.BlockSpec((B * Q, D), lambda i: (0, 0)),
            pl.BlockSpec((B * S, D), lambda i: (0, 0)),
            full, full, full, full,
            pl.BlockSpec(memory_space=pltpu.MemorySpace.HBM),
            pl.BlockSpec(memory_space=pltpu.MemorySpace.HBM),
            full, full,
        ],
        out_specs=pl.BlockSpec((1, Q, D),
                               lambda i: (jnp.maximum(i - 1, 0), 0, 0)),
        out_shape=jax.ShapeDtypeStruct((B, Q, D), jnp.float32),
        scratch_shapes=[
            pltpu.VMEM((H, B * Q, D), jnp.float32),
            pltpu.VMEM((B * H * Q, S), jnp.float32),
            pltpu.VMEM((NSLOT, S, TOKEN_K, D), jnp.float32),
            pltpu.VMEM((NSLOT, S, TOKEN_K, D), jnp.float32),
            pltpu.SemaphoreType.DMA((2, NSLOT)),
        ],
    )(vl, qf, skf, W_q_stat, W_q_token, W_k_stat, W_k_token,
      token_keys, values, W_v, W_o)
    return out


# speedup vs baseline: 1.8300x; 1.1101x over previous
import math

import jax
import jax.numpy as jnp
from jax import lax
from jax.experimental import pallas as pl
from jax.experimental.pallas import tpu as pltpu

B, Q, S, T = 8, 16, 128, 64
D = 256
H = 8
HD = D // H
STAT_K, TOKEN_K = 8, 16
NEG = -1000000.0
INV_SQRT = 1.0 / math.sqrt(HD)
NSLOT = 3


def _kernel(vl_ref, qf_ref, skf_ref, wqs_ref, wqt_ref, wks_ref, wkt_ref,
            tk_hbm, v_hbm, wv_ref, wo_ref, out_ref,
            qk_scr, w_scr, tkbuf, vbuf, sems):
    i = pl.program_id(0)
    nt = (((1,), (1,)), ((), ()))

    def tk_copy(blk, slot):
        return pltpu.make_async_copy(
            tk_hbm.at[pl.ds(blk * S, S), pl.ds(T - TOKEN_K, TOKEN_K), :],
            tkbuf.at[slot], sems.at[0, slot])

    def v_copy(blk, slot):
        return pltpu.make_async_copy(
            v_hbm.at[pl.ds(blk * S, S), pl.ds(T - TOKEN_K, TOKEN_K), :],
            vbuf.at[slot], sems.at[1, slot])

    @pl.when(i == 0)
    def _stat():
        tk_copy(0, 0).start()
        v_copy(0, 0).start()
        tk_copy(1, 1).start()
        v_copy(1, 1).start()

        qf = qf_ref[...]
        qs = jnp.dot(qf, wqs_ref[...], preferred_element_type=jnp.float32)
        qt = jnp.dot(qf, wqt_ref[...], preferred_element_type=jnp.float32)
        ks = jnp.dot(skf_ref[...], wks_ref[...],
                     preferred_element_type=jnp.float32)

        wkt = wkt_ref[...]
        for h in range(H):
            hsl = slice(h * HD, (h + 1) * HD)
            qk_scr[h] = jax.lax.dot_general(
                qt[:, hsl], wkt[:, hsl], nt,
                preferred_element_type=jnp.float32)

        iota_s = jax.lax.broadcasted_iota(jnp.int32, (H * Q, S), 1)
        blocks = []
        for b in range(B):
            qs_b = qs[b * Q:(b + 1) * Q, :]
            ks_b = ks[b * S:(b + 1) * S, :]
            rows_h = [jax.lax.dot_general(
                qs_b[:, h * HD:(h + 1) * HD], ks_b[:, h * HD:(h + 1) * HD],
                nt, preferred_element_type=jnp.float32) for h in range(H)]
            sc_b = jnp.concatenate(rows_h, axis=0) * INV_SQRT
            blocks.append(jnp.where(iota_s < vl_ref[0, b], sc_b, NEG))
        sc = jnp.concatenate(blocks, axis=0)

        iota_f = jax.lax.broadcasted_iota(
            jnp.int32, (B * H * Q, S), 1).astype(jnp.float32)
        cur = sc
        big = jnp.float32(1e9)
        row_max = None
        for k in range(STAT_K):
            m = jnp.max(cur, axis=-1, keepdims=True)
            if k == 0:
                row_max = m
            eq = cur == m
            fidx = jnp.min(jnp.where(eq, iota_f, big), axis=-1, keepdims=True)
            oh = iota_f == fidx
            cur = jnp.where(oh, jnp.float32(-3.0e38), cur)
        e = jnp.where(cur != sc, jnp.exp(sc - row_max), 0.0)
        w_scr[...] = e / jnp.sum(e, axis=-1, keepdims=True)

    @pl.when(i > 0)
    def _token():
        b = i - 1
        slot = lax.rem(b, NSLOT)

        @pl.when(b + 2 < B)
        def _prefetch():
            nslot = lax.rem(b + 2, NSLOT)
            tk_copy(b + 2, nslot).start()
            v_copy(b + 2, nslot).start()

        tk_copy(b, slot).wait()
        v_copy(b, slot).wait()

        qk = qk_scr[:, pl.ds(b * Q, Q), :].reshape(H * Q, D)
        w = w_scr[pl.ds(b * H * Q, H * Q), :]

        tk = tkbuf[slot].reshape(S * TOKEN_K, D)
        tsc = jax.lax.dot_general(tk, qk, nt,
                                  preferred_element_type=jnp.float32)
        sc3 = (tsc * INV_SQRT).reshape(S, TOKEN_K, H * Q)
        mx = jnp.max(sc3, axis=1, keepdims=True)
        te = jnp.exp(sc3 - mx)
        a = te / jnp.sum(te, axis=1, keepdims=True)

        cwf = (a * w.T[:, None, :]).reshape(S * TOKEN_K, H * Q)
        m_acc = jax.lax.dot_general(
            cwf, vbuf[slot].reshape(S * TOKEN_K, D), (((0,), (0,)), ((), ())),
            preferred_element_type=jnp.float32)

        wv = wv_ref[...]
        parts = []
        for h in range(H):
            hsl = slice(h * HD, (h + 1) * HD)
            parts.append(jnp.dot(m_acc[h * Q:(h + 1) * Q, :], wv[:, hsl],
                                 preferred_element_type=jnp.float32))
        out_pre = jnp.concatenate(parts, axis=1)
        out_ref[...] = jnp.dot(out_pre, wo_ref[...],
                               preferred_element_type=jnp.float32)[None]


def kernel(queries, stat_keys, token_keys, values, stat_valid_lens,
           W_q_stat, W_q_token, W_k_stat, W_k_token, W_v, W_o):
    qf = queries.reshape(B * Q, D)
    skf = stat_keys.reshape(B * S, D)
    vl = stat_valid_lens.reshape(1, B).astype(jnp.int32)

    full = pl.BlockSpec((D, D), lambda i: (0, 0))
    out = pl.pallas_call(
        _kernel,
        grid=(B + 1,),
        in_specs=[
            pl.BlockSpec(memory_space=pltpu.SMEM),
            pl.BlockSpec((B * Q, D), lambda i: (0, 0)),
            pl.BlockSpec((B * S, D), lambda i: (0, 0)),
            full, full, full, full,
            pl.BlockSpec(memory_space=pltpu.MemorySpace.HBM),
            pl.BlockSpec(memory_space=pltpu.MemorySpace.HBM),
            full, full,
        ],
        out_specs=pl.BlockSpec((1, Q, D),
                               lambda i: (jnp.maximum(i - 1, 0), 0, 0)),
        out_shape=jax.ShapeDtypeStruct((B, Q, D), jnp.float32),
        scratch_shapes=[
            pltpu.VMEM((H, B * Q, D), jnp.float32),
            pltpu.VMEM((B * H * Q, S), jnp.float32),
            pltpu.VMEM((NSLOT, S, TOKEN_K, D), jnp.float32),
            pltpu.VMEM((NSLOT, S, TOKEN_K, D), jnp.float32),
            pltpu.SemaphoreType.DMA((2, NSLOT)),
        ],
    )(vl, qf, skf, W_q_stat, W_q_token, W_k_stat, W_k_token,
      token_keys, values, W_v, W_o)
    return out
